# Initial kernel scaffold; baseline (speedup 1.0000x reference)
import functools

import jax
import jax.numpy as jnp
from jax import lax
from jax.experimental import pallas as pl
from jax.experimental.pallas import tpu as pltpu
from jax.experimental.pallas import tpu_sc as plsc

ENT = 10000
NENT = 10006
NREL = 500
NH = 20000
NE = 320000
D = 128
B = 4096
L = 16
NC, NS = 2, 16
NW = NC * NS
CH = 128
HCH = 5
NH_PAD = NW * HCH * CH
ECH = 79
NE_PAD = NW * ECH * CH
ACH = 5
NAGG = NS * ACH * CH
DUMP = NENT

_mesh = plsc.VectorSubcoreMesh(core_axis_name="c", subcore_axis_name="s")


def _mul_rows(acc_ref, b_ref):
    def row(r, _):
        for c in range(D // L):
            sl = pl.ds(c * L, L)
            acc_ref[r, sl] = acc_ref[r, sl] * b_ref[r, sl]
        return 0
    lax.fori_loop(0, CH, row, 0)


def _hyper_body(R_h, E_h, relidx_h, entidx_h, cvec_h, h_out,
                idx_v, acc_v, buf_v, cvec_v, sem):
    wid = lax.axis_index("c") * NS + lax.axis_index("s")
    pltpu.sync_copy(cvec_h, cvec_v)

    def chunk(k, _):
        base = (wid * HCH + k) * CH
        pltpu.sync_copy(relidx_h.at[pl.ds(base, CH)], idx_v)
        pltpu.async_copy(R_h.at[idx_v], acc_v, sem).wait()
        for i in range(6):
            pltpu.sync_copy(entidx_h.at[pl.ds(i * NH_PAD + base, CH)], idx_v)
            pltpu.async_copy(E_h.at[idx_v], buf_v, sem).wait()
            _mul_rows(acc_v, buf_v)

        def crow(r, _):
            for c in range(D // L):
                sl = pl.ds(c * L, L)
                acc_v[r, sl] = acc_v[r, sl] * cvec_v[sl]
            return 0
        lax.fori_loop(0, CH, crow, 0)
        pltpu.sync_copy(acc_v, h_out.at[pl.ds(base, CH)])
        return 0
    lax.fori_loop(0, HCH, chunk, 0)


def _edge_body(h_h, E_h, et_h, src_h, dst_h, agg2_out,
               eti_v, srci_v, dsti_v, hbuf_v, ebuf_v, agg_s, sem, sem2):
    cid = lax.axis_index("c")
    sid = lax.axis_index("s")
    wid = cid * NS + sid

    def zrow(r, _):
        for c in range(D // L):
            hbuf_v[r, pl.ds(c * L, L)] = jnp.zeros((L,), jnp.float32)
        return 0
    lax.fori_loop(0, CH, zrow, 0)

    def zchunk(k, _):
        pltpu.sync_copy(hbuf_v, agg_s.at[pl.ds((sid * ACH + k) * CH, CH)])
        return 0
    lax.fori_loop(0, ACH, zchunk, 0)
    plsc.subcore_barrier()

    def chunk(k, _):
        base = (wid * ECH + k) * CH
        pltpu.sync_copy(et_h.at[pl.ds(base, CH)], eti_v)
        pltpu.sync_copy(src_h.at[pl.ds(base, CH)], srci_v)
        pltpu.sync_copy(dst_h.at[pl.ds(base, CH)], dsti_v)
        cp1 = pltpu.async_copy(h_h.at[eti_v], hbuf_v, sem)
        cp2 = pltpu.async_copy(E_h.at[srci_v], ebuf_v, sem2)
        cp1.wait()
        cp2.wait()
        _mul_rows(hbuf_v, ebuf_v)
        pltpu.sync_copy(hbuf_v, agg_s.at[dsti_v], add=True)
        return 0
    lax.fori_loop(0, ECH, chunk, 0)
    plsc.subcore_barrier()

    def ochunk(k, _):
        off = (sid * ACH + k) * CH
        pltpu.sync_copy(agg_s.at[pl.ds(off, CH)], agg2_out.at[cid, pl.ds(off, CH)])
        return 0
    lax.fori_loop(0, ACH, ochunk, 0)


def _post_body(agg2_ref, E_ref, R_ref, w_ref, g_ref, b_ref,
               out_ref, rout_ref, qc_ref):
    agg = agg2_ref[0] + agg2_ref[1]
    pre = agg * 0.5 + E_ref[...] * 0.5
    rows = lax.broadcasted_iota(jnp.int32, (NAGG, 1), 0)
    mask = rows < NENT
    xm = jnp.where(mask, pre, 0.0)
    s1 = jnp.sum(xm, axis=0, keepdims=True)
    s2 = jnp.sum(xm * xm, axis=0, keepdims=True)
    mean = s1 / NENT
    var = s2 / NENT - mean * mean
    inv = lax.rsqrt(var + 1e-5)
    y = jnp.tanh((pre - mean) * inv * g_ref[...] + b_ref[...])
    out_ref[...] = y
    rout_ref[...] = jnp.dot(R_ref[...], w_ref[...],
                            preferred_element_type=jnp.float32)
    qc = (y[ENT:ENT + 1] * y[ENT + 1:ENT + 2] * y[ENT + 2:ENT + 3]
          * y[ENT + 3:ENT + 4] * y[ENT + 4:ENT + 5] * y[ENT + 5:ENT + 6])
    qc_ref[...] = jnp.broadcast_to(qc, (8, D))


def _score_body(out_h, rout_h, qc_h, idx_h, score_out,
                idx_v, acc_v, buf_v, qc_v, score_v, sem):
    wid = lax.axis_index("c") * NS + lax.axis_index("s")
    base0 = wid * CH
    pltpu.sync_copy(qc_h.at[0], qc_v)
    pltpu.sync_copy(idx_h.at[pl.ds(base0, CH)], idx_v)
    pltpu.async_copy(rout_h.at[idx_v], acc_v, sem).wait()
    for i in range(1, 7):
        pltpu.sync_copy(idx_h.at[pl.ds(i * B + base0, CH)], idx_v)
        pltpu.async_copy(out_h.at[idx_v], buf_v, sem).wait()
        _mul_rows(acc_v, buf_v)

    def row(r, _):
        sv = jnp.zeros((L,), jnp.float32)
        for c in range(D // L):
            sl = pl.ds(c * L, L)
            sv = sv + acc_v[r, sl] * qc_v[sl]
        score_v[r] = jnp.sum(sv)
        return 0
    lax.fori_loop(0, CH, row, 0)
    pltpu.sync_copy(score_v, score_out.at[pl.ds(base0, CH)])


_hyper_call = functools.partial(
    pl.kernel,
    out_type=jax.ShapeDtypeStruct((NH_PAD, D), jnp.float32),
    mesh=_mesh,
    scratch_types=[
        pltpu.VMEM((CH,), jnp.int32),
        pltpu.VMEM((CH, D), jnp.float32),
        pltpu.VMEM((CH, D), jnp.float32),
        pltpu.VMEM((D,), jnp.float32),
        pltpu.SemaphoreType.DMA,
    ],
)(_hyper_body)

_edge_call = functools.partial(
    pl.kernel,
    out_type=jax.ShapeDtypeStruct((NC, NAGG, D), jnp.float32),
    mesh=_mesh,
    scratch_types=[
        pltpu.VMEM((CH,), jnp.int32),
        pltpu.VMEM((CH,), jnp.int32),
        pltpu.VMEM((CH,), jnp.int32),
        pltpu.VMEM((CH, D), jnp.float32),
        pltpu.VMEM((CH, D), jnp.float32),
        pltpu.VMEM_SHARED((NAGG, D), jnp.float32),
        pltpu.SemaphoreType.DMA,
        pltpu.SemaphoreType.DMA,
    ],
)(_edge_body)

_post_call = pl.pallas_call(
    _post_body,
    out_shape=[
        jax.ShapeDtypeStruct((NAGG, D), jnp.float32),
        jax.ShapeDtypeStruct((512, D), jnp.float32),
        jax.ShapeDtypeStruct((8, D), jnp.float32),
    ],
)

_score_call = functools.partial(
    pl.kernel,
    out_type=jax.ShapeDtypeStruct((B,), jnp.float32),
    mesh=_mesh,
    scratch_types=[
        pltpu.VMEM((CH,), jnp.int32),
        pltpu.VMEM((CH, D), jnp.float32),
        pltpu.VMEM((CH, D), jnp.float32),
        pltpu.VMEM((D,), jnp.float32),
        pltpu.VMEM((CH,), jnp.float32),
        pltpu.SemaphoreType.DMA,
    ],
)(_score_body)


def kernel(E, R, w_rel, bn_gamma, bn_beta, hyperedge, edge_index, edge_type,
           r_idx, e1_idx, e2_idx, e3_idx, e4_idx, e5_idx, e6_idx):
    f32 = jnp.float32
    i32 = jnp.int32
    E_pad = jnp.zeros((NAGG, D), f32).at[:NENT].set(E)
    R_pad = jnp.zeros((512, D), f32).at[:NREL].set(R)
    cvec = (E[ENT] * E[ENT + 1] * E[ENT + 2]
            * E[ENT + 3] * E[ENT + 4] * E[ENT + 5])

    relidx = jnp.zeros((NH_PAD,), i32).at[:NH].set(hyperedge[:, 0].astype(i32))
    entidx = (jnp.zeros((6, NH_PAD), i32)
              .at[:, :NH].set(hyperedge[:, 1:7].T.astype(i32))
              .reshape(6 * NH_PAD))

    et = jnp.zeros((NE_PAD,), i32).at[:NE].set(edge_type.astype(i32))
    src = jnp.zeros((NE_PAD,), i32).at[:NE].set(edge_index[1].astype(i32))
    dst = (jnp.full((NE_PAD,), DUMP, i32)
           .at[:NE].set(edge_index[0].astype(i32)))

    idxpack = jnp.concatenate([
        r_idx.astype(i32), e1_idx.astype(i32), e2_idx.astype(i32),
        e3_idx.astype(i32), e4_idx.astype(i32), e5_idx.astype(i32),
        e6_idx.astype(i32)])

    h = _hyper_call(R, E_pad, relidx, entidx, cvec)
    agg2 = _edge_call(h, E_pad, et, src, dst)
    out, rout, qc = _post_call(agg2, E_pad, R_pad, w_rel,
                               bn_gamma.reshape(1, D), bn_beta.reshape(1, D))
    score = _score_call(out, rout, qc, idxpack)
    return score


# speedup vs baseline: 3.0052x; 3.0052x over previous
import functools

import jax
import jax.numpy as jnp
from jax import lax
from jax.experimental import pallas as pl
from jax.experimental.pallas import tpu as pltpu
from jax.experimental.pallas import tpu_sc as plsc

ENT = 10000
NENT = 10006
NREL = 500
NH = 20000
NE = 320000
D = 128
B = 4096
L = 16
NC, NS = 2, 16
NW = NC * NS
CH = 128
HCH = 5
NH_PAD = NW * HCH * CH
ECH = 79
NE_PAD = NW * ECH * CH
ACH = 5
NAGG = NS * ACH * CH
DUMP = NENT


def _mul_rows(acc_ref, b_ref):
    def row(r, _):
        for c in range(D // L):
            sl = pl.ds(c * L, L)
            acc_ref[r, sl] = acc_ref[r, sl] * b_ref[r, sl]
        return 0
    lax.fori_loop(0, CH, row, 0)


def _hyper_body(R_h, E_h, relidx_h, entidx_h, cvec_h, h_out,
                idx_v, acc_v, buf_v, cvec_v, sem):
    wid = lax.axis_index("c") * NS + lax.axis_index("s")
    pltpu.sync_copy(cvec_h, cvec_v)

    def chunk(k, _):
        base = (wid * HCH + k) * CH
        pltpu.sync_copy(relidx_h.at[pl.ds(base, CH)], idx_v)
        pltpu.async_copy(R_h.at[idx_v], acc_v, sem).wait()
        for i in range(6):
            pltpu.sync_copy(entidx_h.at[pl.ds(i * NH_PAD + base, CH)], idx_v)
            pltpu.async_copy(E_h.at[idx_v], buf_v, sem).wait()
            _mul_rows(acc_v, buf_v)

        def crow(r, _):
            for c in range(D // L):
                sl = pl.ds(c * L, L)
                acc_v[r, sl] = acc_v[r, sl] * cvec_v[sl]
            return 0
        lax.fori_loop(0, CH, crow, 0)
        pltpu.sync_copy(acc_v, h_out.at[pl.ds(base, CH)])
        return 0
    lax.fori_loop(0, HCH, chunk, 0)


def _edge_body(h_h, E_h, et_h, src_h, dst_h, agg2_out,
               eti_v, srci_v, dsti_v, hbuf_v, ebuf_v, agg_s, sem, sem2):
    cid = lax.axis_index("c")
    sid = lax.axis_index("s")
    wid = cid * NS + sid

    def zrow(r, _):
        for c in range(D // L):
            hbuf_v[r, pl.ds(c * L, L)] = jnp.zeros((L,), jnp.float32)
        return 0
    lax.fori_loop(0, CH, zrow, 0)

    def zchunk(k, _):
        pltpu.sync_copy(hbuf_v, agg_s.at[pl.ds((sid * ACH + k) * CH, CH)])
        return 0
    lax.fori_loop(0, ACH, zchunk, 0)
    plsc.subcore_barrier()

    def chunk(k, _):
        base = (wid * ECH + k) * CH
        pltpu.sync_copy(et_h.at[pl.ds(base, CH)], eti_v)
        pltpu.sync_copy(src_h.at[pl.ds(base, CH)], srci_v)
        pltpu.sync_copy(dst_h.at[pl.ds(base, CH)], dsti_v)
        cp1 = pltpu.async_copy(h_h.at[eti_v], hbuf_v, sem)
        cp2 = pltpu.async_copy(E_h.at[srci_v], ebuf_v, sem2)
        cp1.wait()
        cp2.wait()
        _mul_rows(hbuf_v, ebuf_v)
        pltpu.sync_copy(hbuf_v, agg_s.at[dsti_v], add=True)
        return 0
    lax.fori_loop(0, ECH, chunk, 0)
    plsc.subcore_barrier()

    def ochunk(k, _):
        off = (sid * ACH + k) * CH
        pltpu.sync_copy(agg_s.at[pl.ds(off, CH)], agg2_out.at[cid, pl.ds(off, CH)])
        return 0
    lax.fori_loop(0, ACH, ochunk, 0)


def _post_body(agg2_ref, E_ref, R_ref, w_ref, g_ref, b_ref,
               out_ref, rout_ref, qc_ref):
    agg = agg2_ref[0] + agg2_ref[1]
    pre = agg * 0.5 + E_ref[...] * 0.5
    rows = lax.broadcasted_iota(jnp.int32, (NAGG, 1), 0)
    mask = rows < NENT
    xm = jnp.where(mask, pre, 0.0)
    s1 = jnp.sum(xm, axis=0, keepdims=True)
    s2 = jnp.sum(xm * xm, axis=0, keepdims=True)
    mean = s1 / NENT
    var = s2 / NENT - mean * mean
    inv = lax.rsqrt(var + 1e-5)
    y = jnp.tanh((pre - mean) * inv * g_ref[...] + b_ref[...])
    out_ref[...] = y
    rout_ref[...] = jnp.dot(R_ref[...], w_ref[...],
                            preferred_element_type=jnp.float32)
    qc = (y[ENT:ENT + 1] * y[ENT + 1:ENT + 2] * y[ENT + 2:ENT + 3]
          * y[ENT + 3:ENT + 4] * y[ENT + 4:ENT + 5] * y[ENT + 5:ENT + 6])
    qc_ref[...] = jnp.broadcast_to(qc, (8, D))


def _score_body(out_h, rout_h, idx_h, prod_out,
                idx_v, acc_v, buf_v, sem):
    wid = lax.axis_index("c") * NS + lax.axis_index("s")
    base0 = wid * CH
    pltpu.sync_copy(idx_h.at[pl.ds(base0, CH)], idx_v)
    pltpu.async_copy(rout_h.at[idx_v], acc_v, sem).wait()
    for i in range(1, 7):
        pltpu.sync_copy(idx_h.at[pl.ds(i * B + base0, CH)], idx_v)
        pltpu.async_copy(out_h.at[idx_v], buf_v, sem).wait()
        _mul_rows(acc_v, buf_v)
    pltpu.sync_copy(acc_v, prod_out.at[pl.ds(base0, CH)])


def _final_body(prod_ref, qc_ref, score_ref):
    score_ref[...] = jnp.sum(prod_ref[...] * qc_ref[0:1, :], axis=1)


@functools.cache
def _build_calls():
    mesh = plsc.VectorSubcoreMesh(core_axis_name="c", subcore_axis_name="s",
                                  num_cores=NC, num_subcores=NS)
    hyper_call = functools.partial(
        pl.kernel,
        out_type=jax.ShapeDtypeStruct((NH_PAD, D), jnp.float32),
        mesh=mesh,
        scratch_types=[
            pltpu.VMEM((CH,), jnp.int32),
            pltpu.VMEM((CH, D), jnp.float32),
            pltpu.VMEM((CH, D), jnp.float32),
            pltpu.VMEM((D,), jnp.float32),
            pltpu.SemaphoreType.DMA,
        ],
    )(_hyper_body)

    edge_call = functools.partial(
        pl.kernel,
        out_type=jax.ShapeDtypeStruct((NC, NAGG, D), jnp.float32),
        mesh=mesh,
        scratch_types=[
            pltpu.VMEM((CH,), jnp.int32),
            pltpu.VMEM((CH,), jnp.int32),
            pltpu.VMEM((CH,), jnp.int32),
            pltpu.VMEM((CH, D), jnp.float32),
            pltpu.VMEM((CH, D), jnp.float32),
            pltpu.VMEM_SHARED((NAGG, D), jnp.float32),
            pltpu.SemaphoreType.DMA,
            pltpu.SemaphoreType.DMA,
        ],
    )(_edge_body)

    post_call = pl.pallas_call(
        _post_body,
        out_shape=[
            jax.ShapeDtypeStruct((NAGG, D), jnp.float32),
            jax.ShapeDtypeStruct((512, D), jnp.float32),
            jax.ShapeDtypeStruct((8, D), jnp.float32),
        ],
    )

    score_call = functools.partial(
        pl.kernel,
        out_type=jax.ShapeDtypeStruct((B, D), jnp.float32),
        mesh=mesh,
        scratch_types=[
            pltpu.VMEM((CH,), jnp.int32),
            pltpu.VMEM((CH, D), jnp.float32),
            pltpu.VMEM((CH, D), jnp.float32),
            pltpu.SemaphoreType.DMA,
        ],
    )(_score_body)

    final_call = pl.pallas_call(
        _final_body,
        out_shape=jax.ShapeDtypeStruct((B,), jnp.float32),
    )
    return hyper_call, edge_call, post_call, score_call, final_call


def kernel(E, R, w_rel, bn_gamma, bn_beta, hyperedge, edge_index, edge_type,
           r_idx, e1_idx, e2_idx, e3_idx, e4_idx, e5_idx, e6_idx):
    f32 = jnp.float32
    i32 = jnp.int32
    E_pad = jnp.zeros((NAGG, D), f32).at[:NENT].set(E)
    R_pad = jnp.zeros((512, D), f32).at[:NREL].set(R)
    cvec = (E[ENT] * E[ENT + 1] * E[ENT + 2]
            * E[ENT + 3] * E[ENT + 4] * E[ENT + 5])

    relidx = jnp.zeros((NH_PAD,), i32).at[:NH].set(hyperedge[:, 0].astype(i32))
    entidx = (jnp.zeros((6, NH_PAD), i32)
              .at[:, :NH].set(hyperedge[:, 1:7].T.astype(i32))
              .reshape(6 * NH_PAD))

    et = jnp.zeros((NE_PAD,), i32).at[:NE].set(edge_type.astype(i32))
    src = jnp.zeros((NE_PAD,), i32).at[:NE].set(edge_index[1].astype(i32))
    dst = (jnp.full((NE_PAD,), DUMP, i32)
           .at[:NE].set(edge_index[0].astype(i32)))

    idxpack = jnp.concatenate([
        r_idx.astype(i32), e1_idx.astype(i32), e2_idx.astype(i32),
        e3_idx.astype(i32), e4_idx.astype(i32), e5_idx.astype(i32),
        e6_idx.astype(i32)])

    hyper_call, edge_call, post_call, score_call, final_call = _build_calls()
    h = hyper_call(R, E_pad, relidx, entidx, cvec)
    agg2 = edge_call(h, E_pad, et, src, dst)
    out, rout, qc = post_call(agg2, E_pad, R_pad, w_rel,
                              bn_gamma.reshape(1, D), bn_beta.reshape(1, D))
    prod = score_call(out, rout, idxpack)
    score = final_call(prod, qc)
    return score


# speedup vs baseline: 3.4109x; 1.1350x over previous
import functools

import jax
import jax.numpy as jnp
from jax import lax
from jax.experimental import pallas as pl
from jax.experimental.pallas import tpu as pltpu
from jax.experimental.pallas import tpu_sc as plsc

ENT = 10000
NENT = 10006
NREL = 500
NH = 20000
NE = 320000
D = 128
B = 4096
L = 16
NC, NS = 2, 16
NW = NC * NS
CH = 128
HCH = 5
NH_PAD = NW * HCH * CH
CHB = 64
ECH = 160
NPAIR = ECH // 2
NE_PAD = NW * ECH * CHB
ACH = 5
NAGG = NS * ACH * CH
DUMP = NENT


def _mul_rows(acc_ref, b_ref, n=CH):
    def row(r, _):
        for c in range(D // L):
            sl = pl.ds(c * L, L)
            acc_ref[r, sl] = acc_ref[r, sl] * b_ref[r, sl]
        return 0
    lax.fori_loop(0, n, row, 0)


def _mul_rows_cvec(acc_ref, b_ref, cvec_ref):
    def row(r, _):
        for c in range(D // L):
            sl = pl.ds(c * L, L)
            acc_ref[r, sl] = acc_ref[r, sl] * b_ref[r, sl] * cvec_ref[sl]
        return 0
    lax.fori_loop(0, CH, row, 0)


def _hyper_body(R_h, E_h, aidx_h, cvec_h, h_out,
                aidx_v, acc0_v, acc1_v, buf0_v, buf1_v, cvec_v,
                semr, semb0, semb1, semst0, semst1):
    wid = lax.axis_index("c") * NS + lax.axis_index("s")
    pltpu.sync_copy(cvec_h, cvec_v)
    pltpu.sync_copy(aidx_h.at[wid], aidx_v)
    accs = (acc0_v, acc1_v)
    bufs = (buf0_v, buf1_v)
    semb = (semb0, semb1)
    semst = (semst0, semst1)
    store_cp = [None, None]
    for c in range(HCH):
        p = c % 2
        acc = accs[p]
        if store_cp[p] is not None:
            store_cp[p].wait()
        cpr = pltpu.async_copy(R_h.at[aidx_v.at[c, 0]], acc, semr)
        cps = [None, None]
        cps[0] = pltpu.async_copy(E_h.at[aidx_v.at[c, 1]], bufs[0], semb[0])
        cpr.wait()
        for i in range(1, 7):
            cur = (i - 1) % 2
            if i < 6:
                cps[i % 2] = pltpu.async_copy(
                    E_h.at[aidx_v.at[c, i + 1]], bufs[i % 2], semb[i % 2])
            cps[cur].wait()
            if i < 6:
                _mul_rows(acc, bufs[cur])
            else:
                _mul_rows_cvec(acc, bufs[cur], cvec_v)
        base = (wid * HCH + c) * CH
        store_cp[p] = pltpu.async_copy(acc, h_out.at[pl.ds(base, CH)], semst[p])
    for p in range(2):
        if store_cp[p] is not None:
            store_cp[p].wait()


def _edge_body(h_h, E_h, etsd_h, agg2_out,
               pidx_v, dstb_v, hbuf0_v, hbuf1_v, ebuf0_v, ebuf1_v, agg_s,
               semh0, semh1, seme0, seme1):
    cid = lax.axis_index("c")
    sid = lax.axis_index("s")
    wid = cid * NS + sid

    def zrow(r, _):
        for c in range(D // L):
            hbuf0_v[r, pl.ds(c * L, L)] = jnp.zeros((L,), jnp.float32)
        return 0
    lax.fori_loop(0, CHB, zrow, 0)

    nz = NAGG // (NS * CHB)
    def zchunk(k, _):
        pltpu.sync_copy(hbuf0_v, agg_s.at[pl.ds((sid * nz + k) * CHB, CHB)])
        return 0
    lax.fori_loop(0, nz, zchunk, 0)
    plsc.subcore_barrier()

    pltpu.sync_copy(etsd_h.at[wid, 0], pidx_v)
    pltpu.async_copy(h_h.at[pidx_v.at[0, 0]], hbuf0_v, semh0)
    pltpu.async_copy(E_h.at[pidx_v.at[0, 1]], ebuf0_v, seme0)

    def pair(p, _):
        pltpu.async_copy(h_h.at[pidx_v.at[1, 0]], hbuf1_v, semh1)
        pltpu.async_copy(E_h.at[pidx_v.at[1, 1]], ebuf1_v, seme1)
        for c in range(CHB // L):
            dstb_v[pl.ds(c * L, L)] = pidx_v[1, 2, pl.ds(c * L, L)]
        pltpu.make_async_copy(h_h.at[pidx_v.at[0, 0]], hbuf0_v, semh0).wait()
        pltpu.make_async_copy(E_h.at[pidx_v.at[0, 1]], ebuf0_v, seme0).wait()
        _mul_rows(hbuf0_v, ebuf0_v, CHB)
        pltpu.sync_copy(hbuf0_v, agg_s.at[pidx_v.at[0, 2]], add=True)

        @pl.when(p < NPAIR - 1)
        def _():
            pltpu.sync_copy(etsd_h.at[wid, p + 1], pidx_v)
            pltpu.async_copy(h_h.at[pidx_v.at[0, 0]], hbuf0_v, semh0)
            pltpu.async_copy(E_h.at[pidx_v.at[0, 1]], ebuf0_v, seme0)

        pltpu.make_async_copy(h_h.at[pidx_v.at[1, 0]], hbuf1_v, semh1).wait()
        pltpu.make_async_copy(E_h.at[pidx_v.at[1, 1]], ebuf1_v, seme1).wait()
        _mul_rows(hbuf1_v, ebuf1_v, CHB)
        pltpu.sync_copy(hbuf1_v, agg_s.at[dstb_v], add=True)
        return 0
    lax.fori_loop(0, NPAIR, pair, 0)
    plsc.subcore_barrier()

    def ochunk(k, _):
        off = (sid * ACH + k) * CH
        pltpu.sync_copy(agg_s.at[pl.ds(off, CH)], agg2_out.at[cid, pl.ds(off, CH)])
        return 0
    lax.fori_loop(0, ACH, ochunk, 0)


def _post_body(agg2_ref, E_ref, R_ref, w_ref, g_ref, b_ref,
               out_ref, rout_ref, qc_ref):
    agg = agg2_ref[0] + agg2_ref[1]
    pre = agg * 0.5 + E_ref[...] * 0.5
    rows = lax.broadcasted_iota(jnp.int32, (NAGG, 1), 0)
    mask = rows < NENT
    xm = jnp.where(mask, pre, 0.0)
    s1 = jnp.sum(xm, axis=0, keepdims=True)
    s2 = jnp.sum(xm * xm, axis=0, keepdims=True)
    mean = s1 / NENT
    var = s2 / NENT - mean * mean
    inv = lax.rsqrt(var + 1e-5)
    y = jnp.tanh((pre - mean) * inv * g_ref[...] + b_ref[...])
    out_ref[...] = y
    rout_ref[...] = jnp.dot(R_ref[...], w_ref[...],
                            preferred_element_type=jnp.float32)
    qc = (y[ENT:ENT + 1] * y[ENT + 1:ENT + 2] * y[ENT + 2:ENT + 3]
          * y[ENT + 3:ENT + 4] * y[ENT + 4:ENT + 5] * y[ENT + 5:ENT + 6])
    qc_ref[...] = jnp.broadcast_to(qc, (8, D))


def _score_body(out_h, rout_h, idx_h, prod_out,
                idx_v, acc_v, buf0_v, buf1_v, semr, semb0, semb1):
    wid = lax.axis_index("c") * NS + lax.axis_index("s")
    base0 = wid * CH
    pltpu.sync_copy(idx_h.at[wid], idx_v)
    bufs = (buf0_v, buf1_v)
    semb = (semb0, semb1)
    cpr = pltpu.async_copy(rout_h.at[idx_v.at[0]], acc_v, semr)
    cps = [None, None]
    cps[0] = pltpu.async_copy(out_h.at[idx_v.at[1]], bufs[0], semb[0])
    cpr.wait()
    for i in range(1, 7):
        cur = (i - 1) % 2
        if i < 6:
            cps[i % 2] = pltpu.async_copy(
                out_h.at[idx_v.at[i + 1]], bufs[i % 2], semb[i % 2])
        cps[cur].wait()
        _mul_rows(acc_v, bufs[cur])
    pltpu.sync_copy(acc_v, prod_out.at[pl.ds(base0, CH)])


def _final_body(prod_ref, qc_ref, score_ref):
    score_ref[...] = jnp.sum(prod_ref[...] * qc_ref[0:1, :], axis=1)


@functools.cache
def _build_calls():
    mesh = plsc.VectorSubcoreMesh(core_axis_name="c", subcore_axis_name="s",
                                  num_cores=NC, num_subcores=NS)
    hyper_call = functools.partial(
        pl.kernel,
        out_type=jax.ShapeDtypeStruct((NH_PAD, D), jnp.float32),
        mesh=mesh,
        scratch_types=[
            pltpu.VMEM((HCH, 7, CH), jnp.int32),
            pltpu.VMEM((CH, D), jnp.float32),
            pltpu.VMEM((CH, D), jnp.float32),
            pltpu.VMEM((CH, D), jnp.float32),
            pltpu.VMEM((CH, D), jnp.float32),
            pltpu.VMEM((D,), jnp.float32),
            pltpu.SemaphoreType.DMA,
            pltpu.SemaphoreType.DMA,
            pltpu.SemaphoreType.DMA,
            pltpu.SemaphoreType.DMA,
            pltpu.SemaphoreType.DMA,
        ],
    )(_hyper_body)

    edge_call = functools.partial(
        pl.kernel,
        out_type=jax.ShapeDtypeStruct((NC, NAGG, D), jnp.float32),
        mesh=mesh,
        scratch_types=[
            pltpu.VMEM((2, 3, CHB), jnp.int32),
            pltpu.VMEM((CHB,), jnp.int32),
            pltpu.VMEM((CHB, D), jnp.float32),
            pltpu.VMEM((CHB, D), jnp.float32),
            pltpu.VMEM((CHB, D), jnp.float32),
            pltpu.VMEM((CHB, D), jnp.float32),
            pltpu.VMEM_SHARED((NAGG, D), jnp.float32),
            pltpu.SemaphoreType.DMA,
            pltpu.SemaphoreType.DMA,
            pltpu.SemaphoreType.DMA,
            pltpu.SemaphoreType.DMA,
        ],
    )(_edge_body)

    post_call = pl.pallas_call(
        _post_body,
        out_shape=[
            jax.ShapeDtypeStruct((NAGG, D), jnp.float32),
            jax.ShapeDtypeStruct((512, D), jnp.float32),
            jax.ShapeDtypeStruct((8, D), jnp.float32),
        ],
    )

    score_call = functools.partial(
        pl.kernel,
        out_type=jax.ShapeDtypeStruct((B, D), jnp.float32),
        mesh=mesh,
        scratch_types=[
            pltpu.VMEM((7, CH), jnp.int32),
            pltpu.VMEM((CH, D), jnp.float32),
            pltpu.VMEM((CH, D), jnp.float32),
            pltpu.VMEM((CH, D), jnp.float32),
            pltpu.SemaphoreType.DMA,
            pltpu.SemaphoreType.DMA,
            pltpu.SemaphoreType.DMA,
        ],
    )(_score_body)

    final_call = pl.pallas_call(
        _final_body,
        out_shape=jax.ShapeDtypeStruct((B,), jnp.float32),
    )
    return hyper_call, edge_call, post_call, score_call, final_call


def kernel(E, R, w_rel, bn_gamma, bn_beta, hyperedge, edge_index, edge_type,
           r_idx, e1_idx, e2_idx, e3_idx, e4_idx, e5_idx, e6_idx):
    f32 = jnp.float32
    i32 = jnp.int32
    E_pad = jnp.zeros((NAGG, D), f32).at[:NENT].set(E)
    R_pad = jnp.zeros((512, D), f32).at[:NREL].set(R)
    cvec = (E[ENT] * E[ENT + 1] * E[ENT + 2]
            * E[ENT + 3] * E[ENT + 4] * E[ENT + 5])

    relidx = (jnp.zeros((NH_PAD,), i32).at[:NH].set(hyperedge[:, 0].astype(i32))
              .reshape(NW, HCH, 1, CH))
    entidx = (jnp.zeros((6, NH_PAD), i32)
              .at[:, :NH].set(hyperedge[:, 1:7].T.astype(i32))
              .reshape(6, NW, HCH, CH).transpose(1, 2, 0, 3))
    aidx = jnp.concatenate([relidx, entidx], axis=2)

    et = jnp.zeros((NE_PAD,), i32).at[:NE].set(edge_type.astype(i32))
    src = jnp.zeros((NE_PAD,), i32).at[:NE].set(edge_index[1].astype(i32))
    dst = (jnp.full((NE_PAD,), DUMP, i32)
           .at[:NE].set(edge_index[0].astype(i32)))
    etsd = (jnp.stack([et.reshape(NW, ECH, CHB), src.reshape(NW, ECH, CHB),
                       dst.reshape(NW, ECH, CHB)], axis=2)
            .reshape(NW, NPAIR, 2, 3, CHB))

    idxpack = (jnp.stack([
        r_idx.astype(i32), e1_idx.astype(i32), e2_idx.astype(i32),
        e3_idx.astype(i32), e4_idx.astype(i32), e5_idx.astype(i32),
        e6_idx.astype(i32)]).reshape(7, NW, CH).transpose(1, 0, 2))

    hyper_call, edge_call, post_call, score_call, final_call = _build_calls()
    h = hyper_call(R, E_pad, aidx, cvec)
    agg2 = edge_call(h, E_pad, etsd)
    out, rout, qc = post_call(agg2, E_pad, R_pad, w_rel,
                              bn_gamma.reshape(1, D), bn_beta.reshape(1, D))
    prod = score_call(out, rout, idxpack)
    score = final_call(prod, qc)
    return score
